# baseline (speedup 1.0000x reference)
import functools

import jax
import jax.numpy as jnp
from jax import lax
from jax.experimental import pallas as pl
from jax.experimental.pallas import tpu as pltpu
from jax.experimental.pallas import tpu_sc as plsc

N_NODES = 10000
N_EDGES = 320000
D_FEAT = 128
N_HIDDEN = 128
N_CLASSES = 64

NC = 2
NS = 16
NW = NC * NS
LANES = 16
CHUNK = 112
CPT = 90
N_CHUNKS_PAD = NW * CPT
N_EDGES_PAD = N_CHUNKS_PAD * CHUNK
N_PAD = 10240
ROWS_PER_TILE = N_PAD // NS
DUMP_ROW = N_PAD - 1
LAST = CPT - 1


def _sc_round_body(ft_hbm, src_hbm, dst_hbm, nrm_hbm, zero_hbm, out_hbm,
                   si_a, si_b, si_c, di_a, di_b, di_c, nr_a, nr_b, nr_c,
                   rows_a, rows_b, rows_c, acc,
                   gs_a, gs_b, gs_c, is_a, is_b, is_c,
                   ss_a, ss_b, ss_c, ds_a, ds_b, ds_c):
    cid = lax.axis_index("c")
    sid = lax.axis_index("s")
    wid = sid * NC + cid
    tile_base = pl.multiple_of(sid * ROWS_PER_TILE, ROWS_PER_TILE)
    cbase = wid * CPT

    si = (si_a, si_b, si_c)
    di = (di_a, di_b, di_c)
    nr = (nr_a, nr_b, nr_c)
    rows = (rows_a, rows_b, rows_c)
    gs = (gs_a, gs_b, gs_c)
    isem = (is_a, is_b, is_c)
    ssem = (ss_a, ss_b, ss_c)
    dsem = (ds_a, ds_b, ds_c)

    pltpu.sync_copy(zero_hbm, acc.at[pl.ds(tile_base, ROWS_PER_TILE)])
    plsc.subcore_barrier()

    def iload(c, x):
        base = (cbase + c) * CHUNK
        pltpu.async_copy(src_hbm.at[pl.ds(base, CHUNK)], si[x], isem[x])
        pltpu.async_copy(nrm_hbm.at[pl.ds(base, CHUNK)], nr[x], isem[x])

    def iwait(x):
        pltpu.make_async_copy(src_hbm.at[pl.ds(0, CHUNK)], si[x], isem[x]).wait()
        pltpu.make_async_copy(nrm_hbm.at[pl.ds(0, CHUNK)], nr[x], isem[x]).wait()

    def dload(c, x):
        base = (cbase + c) * CHUNK
        pltpu.async_copy(dst_hbm.at[pl.ds(base, CHUNK)], di[x], dsem[x])

    def dwait(x):
        pltpu.make_async_copy(dst_hbm.at[pl.ds(0, CHUNK)], di[x], dsem[x]).wait()

    def scatter(x):
        pltpu.async_copy(rows[x], acc.at[di[x]], ssem[x], add=True)

    def swait(x):
        pltpu.make_async_copy(rows[x], acc.at[di[x]], ssem[x]).wait()

    def gather(x):
        pltpu.async_copy(ft_hbm.at[si[x]], rows[x], gs[x])

    def gwait(x):
        pltpu.make_async_copy(ft_hbm.at[si[x]], rows[x], gs[x]).wait()

    def scale(x):
        buf = rows[x]
        nv_ref = nr[x]
        for t in range(CHUNK // LANES):
            nv = nv_ref[pl.ds(t * LANES, LANES)]
            for el in range(LANES):
                e = t * LANES + el
                s = nv[el]
                for j in range(D_FEAT // LANES):
                    slc = pl.ds(j * LANES, LANES)
                    buf[e, slc] = buf[e, slc] * s

    iload(0, 0)
    iload(1, 1)
    iload(2, 2)
    dload(0, 0)
    dload(1, 1)
    iwait(0)
    gather(0)
    iwait(1)
    gather(1)

    @pl.loop(0, CPT // 3)
    def _(tr):
        c0 = tr * 3
        for j in range(3):
            c = c0 + j
            x = j
            z = (j + 2) % 3
            dwait(x)
            gwait(x)
            scale(x)
            scatter(x)

            @pl.when(c + 3 <= LAST)
            def _():
                iload(c + 3, x)

            @pl.when(c + 2 <= LAST)
            def _():
                @pl.when(c >= 1)
                def _():
                    swait(z)
                dload(c + 2, z)
                iwait(z)
                gather(z)

    swait((LAST - 2) % 3)
    swait((LAST - 1) % 3)
    swait(LAST % 3)

    plsc.subcore_barrier()

    pltpu.sync_copy(acc.at[pl.ds(tile_base, ROWS_PER_TILE)],
                    out_hbm.at[cid, pl.ds(tile_base, ROWS_PER_TILE)])


def _sc_round(ft, src, dst, nrm, zeros):
    mesh = plsc.VectorSubcoreMesh(core_axis_name="c", subcore_axis_name="s")
    kern = pl.kernel(
        _sc_round_body,
        out_type=jax.ShapeDtypeStruct((NC, N_PAD, D_FEAT), jnp.float32),
        mesh=mesh,
        scratch_types=(
            [pltpu.VMEM((CHUNK,), jnp.int32)] * 3
            + [pltpu.VMEM((CHUNK,), jnp.int32)] * 3
            + [pltpu.VMEM((CHUNK,), jnp.float32)] * 3
            + [pltpu.VMEM((CHUNK, D_FEAT), jnp.float32)] * 3
            + [pltpu.VMEM_SHARED((N_PAD, D_FEAT), jnp.float32)]
            + [pltpu.SemaphoreType.DMA] * 12
        ),
    )
    return kern(ft, src, dst, nrm, zeros)


def _merge_body(p_ref, o_ref):
    o_ref[...] = p_ref[0, :N_NODES] + p_ref[1, :N_NODES]


def _merge(parts):
    return pl.pallas_call(
        _merge_body,
        out_shape=jax.ShapeDtypeStruct((N_NODES, D_FEAT), jnp.float32),
    )(parts)


def _mlp_body(p_ref, w1_ref, b1_ref, g_ref, be_ref, w2_ref, b2_ref, o_ref):
    ft = p_ref[0, :N_NODES] + p_ref[1, :N_NODES]
    h = lax.dot_general(ft, w1_ref[...], (((1,), (1,)), ((), ())),
                        precision=lax.Precision.HIGHEST,
                        preferred_element_type=jnp.float32)
    h = h + b1_ref[...][None, :]
    mean = jnp.mean(h, axis=0)
    var = jnp.mean(jnp.square(h), axis=0) - jnp.square(mean)
    h = (h - mean[None, :]) * (g_ref[...] / jnp.sqrt(var + 1e-5))[None, :]
    h = h + be_ref[...][None, :]
    h = jnp.maximum(h, 0.0)
    o = lax.dot_general(h, w2_ref[...], (((1,), (1,)), ((), ())),
                        precision=lax.Precision.HIGHEST,
                        preferred_element_type=jnp.float32)
    o_ref[...] = o + b2_ref[...][None, :]


def _mlp(parts, W1, b1, gamma, beta, W2, b2):
    return pl.pallas_call(
        _mlp_body,
        out_shape=jax.ShapeDtypeStruct((N_NODES, N_CLASSES), jnp.float32),
    )(parts, W1, b1, gamma, beta, W2, b2)


def kernel(feat, edge_index, gcn_norm, W1, b1, gamma, beta, W2, b2):
    pad = N_EDGES_PAD - N_EDGES
    src = jnp.concatenate(
        [edge_index[0].astype(jnp.int32), jnp.zeros((pad,), jnp.int32)])
    dst = jnp.concatenate(
        [edge_index[1].astype(jnp.int32),
         jnp.full((pad,), DUMP_ROW, jnp.int32)])
    nrm = jnp.concatenate([gcn_norm, jnp.zeros((pad,), jnp.float32)])
    zeros = jnp.zeros((ROWS_PER_TILE, D_FEAT), jnp.float32)

    parts = _sc_round(feat, src, dst, nrm, zeros)
    for _ in range(2):
        ft = _merge(parts)
        parts = _sc_round(ft, src, dst, nrm, zeros)
    return _mlp(parts, W1, b1, gamma, beta, W2, b2)
